# baseline (speedup 1.0000x reference)
import functools

import jax
import jax.numpy as jnp
from jax.experimental import pallas as pl
from jax.experimental.pallas import tpu as pltpu

_NSEG = 16


def _pool_kernel(seg_ref, xa_ref, xb_ref, wt_ref, out_ref, m_ref, d_ref, *,
                 nb):
    i = pl.program_id(0)
    nseg = d_ref.shape[0]

    @pl.when(i == 0)
    def _init():
        m_ref[...] = jnp.full(m_ref.shape, -1e30, jnp.float32)
        d_ref[...] = jnp.zeros(d_ref.shape, jnp.float32)
        out_ref[...] = jnp.zeros(out_ref.shape, jnp.float32)

    ids = seg_ref[0]
    rows = xa_ref.shape[0]
    dh = xa_ref.shape[1]
    wt = wt_ref[...]

    xa = xa_ref[...].astype(jnp.bfloat16)
    xb = xb_ref[...].astype(jnp.bfloat16)

    sa = jax.lax.dot_general(
        wt[:, :dh], xa, (((1,), (1,)), ((), ())),
        preferred_element_type=jnp.float32)
    sb = jax.lax.dot_general(
        wt[:, dh:], xb, (((1,), (1,)), ((), ())),
        preferred_element_type=jnp.float32)
    scores = sa + sb

    bm = jnp.max(scores).reshape(1, 1)
    m_old = m_ref[...]
    m_new = jnp.maximum(m_old, bm)
    alpha = jnp.exp(m_old - m_new)
    beta = jnp.exp(bm - m_new)

    e = jnp.exp(scores - bm)
    sub = jax.lax.broadcasted_iota(jnp.int32, (nseg, rows), 0)
    we = jnp.where(sub == ids, e, 0.0).astype(jnp.bfloat16)

    ones = jnp.ones((rows, 1), jnp.bfloat16)
    dsum = jax.lax.dot_general(
        we, ones, (((1,), (0,)), ((), ())),
        preferred_element_type=jnp.float32)
    ca = jax.lax.dot_general(
        we, xa, (((1,), (0,)), ((), ())),
        preferred_element_type=jnp.float32)
    cb = jax.lax.dot_general(
        we, xb, (((1,), (0,)), ((), ())),
        preferred_element_type=jnp.float32)

    m_ref[...] = m_new
    d_ref[...] = alpha * d_ref[...] + beta * dsum
    out_ref[:, :dh] = alpha * out_ref[:, :dh] + beta * ca
    out_ref[:, dh:] = alpha * out_ref[:, dh:] + beta * cb

    @pl.when(i == nb - 1)
    def _finish():
        d = d_ref[...]
        denom = jnp.where(d > 0.0, d, 1.0)
        out_ref[...] = out_ref[...] / denom


@jax.jit
def _attn_pool(x, segment_ids, W):
    n, d = x.shape
    rows = 1024
    nb = n // rows
    dh = d // 2
    ids = segment_ids.astype(jnp.int32).reshape(nb, 1, rows)
    wt = W.reshape(1, d).astype(jnp.bfloat16)
    return pl.pallas_call(
        functools.partial(_pool_kernel, nb=nb),
        grid=(nb,),
        in_specs=[
            pl.BlockSpec((1, 1, rows), lambda i: (i, 0, 0)),
            pl.BlockSpec((rows, dh), lambda i: (i, 0)),
            pl.BlockSpec((rows, dh), lambda i: (i, 1)),
            pl.BlockSpec((1, d), lambda i: (0, 0)),
        ],
        out_specs=pl.BlockSpec((_NSEG, d), lambda i: (0, 0)),
        out_shape=jax.ShapeDtypeStruct((_NSEG, d), jnp.float32),
        scratch_shapes=[
            pltpu.VMEM((1, 1), jnp.float32),
            pltpu.VMEM((_NSEG, 1), jnp.float32),
        ],
        compiler_params=pltpu.CompilerParams(
            dimension_semantics=("arbitrary",)),
    )(ids, x, x, wt)


def kernel(x, segment_ids, W):
    return _attn_pool(x, segment_ids, W)


# speedup vs baseline: 1.1196x; 1.1196x over previous
import functools

import jax
import jax.numpy as jnp
from jax.experimental import pallas as pl
from jax.experimental.pallas import tpu as pltpu

_NSEG = 16


def _pool_kernel(seg_ref, xa_ref, xb_ref, wt_ref, out_ref, xbf_ref, we_ref,
                 bm_ref, m_ref, d_ref, *, nb):
    i = pl.program_id(0)
    nseg = d_ref.shape[0]
    rows = xa_ref.shape[0]
    dh = xa_ref.shape[1]
    slot = jax.lax.rem(i, 2)
    prev = 1 - slot

    @pl.when(i == 0)
    def _init():
        m_ref[...] = jnp.full(m_ref.shape, -1e30, jnp.float32)
        d_ref[...] = jnp.zeros(d_ref.shape, jnp.float32)
        out_ref[...] = jnp.zeros(out_ref.shape, jnp.float32)

    @pl.when(i < nb)
    def _stage():
        ids = seg_ref[0]
        wt = wt_ref[...]

        xa = xa_ref[...].astype(jnp.bfloat16)
        xb = xb_ref[...].astype(jnp.bfloat16)
        xbf_ref[slot, :, :dh] = xa
        xbf_ref[slot, :, dh:] = xb

        sa = jax.lax.dot_general(
            wt[:, :dh], xa, (((1,), (1,)), ((), ())),
            preferred_element_type=jnp.float32)
        sb = jax.lax.dot_general(
            wt[:, dh:], xb, (((1,), (1,)), ((), ())),
            preferred_element_type=jnp.float32)
        scores = sa + sb

        bm = jnp.max(scores).reshape(1, 1)
        bm_ref[pl.ds(slot, 1), :] = bm
        e = jnp.exp(scores - bm)
        sub = jax.lax.broadcasted_iota(jnp.int32, (nseg, rows), 0)
        we_ref[slot] = jnp.where(sub == ids, e, 0.0).astype(jnp.bfloat16)

    @pl.when(i > 0)
    def _pool():
        we = we_ref[prev]
        xc = xbf_ref[prev]

        bm = bm_ref[pl.ds(prev, 1), :]
        m_old = m_ref[...]
        m_new = jnp.maximum(m_old, bm)
        alpha = jnp.exp(m_old - m_new)
        beta = jnp.exp(bm - m_new)

        ones = jnp.ones((rows, 1), jnp.bfloat16)
        dsum = jax.lax.dot_general(
            we, ones, (((1,), (0,)), ((), ())),
            preferred_element_type=jnp.float32)
        contrib = jax.lax.dot_general(
            we, xc, (((1,), (0,)), ((), ())),
            preferred_element_type=jnp.float32)

        m_ref[...] = m_new
        d_ref[...] = alpha * d_ref[...] + beta * dsum
        out_ref[...] = alpha * out_ref[...] + beta * contrib

    @pl.when(i == nb)
    def _finish():
        d = d_ref[...]
        denom = jnp.where(d > 0.0, d, 1.0)
        out_ref[...] = out_ref[...] / denom


@jax.jit
def _attn_pool(x, segment_ids, W):
    n, d = x.shape
    rows = 2048
    nb = n // rows
    dh = d // 2
    ids = segment_ids.astype(jnp.int32).reshape(nb, 1, rows)
    wt = W.reshape(1, d).astype(jnp.bfloat16)
    last = nb - 1
    return pl.pallas_call(
        functools.partial(_pool_kernel, nb=nb),
        grid=(nb + 1,),
        in_specs=[
            pl.BlockSpec((1, 1, rows), lambda i: (jnp.minimum(i, last), 0, 0)),
            pl.BlockSpec((rows, dh), lambda i: (jnp.minimum(i, last), 0)),
            pl.BlockSpec((rows, dh), lambda i: (jnp.minimum(i, last), 1)),
            pl.BlockSpec((1, d), lambda i: (0, 0)),
        ],
        out_specs=pl.BlockSpec((_NSEG, d), lambda i: (0, 0)),
        out_shape=jax.ShapeDtypeStruct((_NSEG, d), jnp.float32),
        scratch_shapes=[
            pltpu.VMEM((2, rows, d), jnp.bfloat16),
            pltpu.VMEM((2, _NSEG, rows), jnp.bfloat16),
            pltpu.VMEM((2, 1), jnp.float32),
            pltpu.VMEM((1, 1), jnp.float32),
            pltpu.VMEM((_NSEG, 1), jnp.float32),
        ],
        compiler_params=pltpu.CompilerParams(
            dimension_semantics=("arbitrary",)),
    )(ids, x, x, wt)


def kernel(x, segment_ids, W):
    return _attn_pool(x, segment_ids, W)


# speedup vs baseline: 1.1506x; 1.0277x over previous
import functools

import jax
import jax.numpy as jnp
from jax.experimental import pallas as pl
from jax.experimental.pallas import tpu as pltpu

_NSEG = 16


def _pool_kernel(seg_ref, xa_ref, xb_ref, wt_ref, out_ref, m_ref, d_ref, *,
                 nb):
    i = pl.program_id(0)
    nseg = d_ref.shape[0]

    @pl.when(i == 0)
    def _init():
        m_ref[...] = jnp.full(m_ref.shape, -1e30, jnp.float32)
        d_ref[...] = jnp.zeros(d_ref.shape, jnp.float32)
        out_ref[...] = jnp.zeros(out_ref.shape, jnp.float32)

    ids = seg_ref[0]
    rows = xa_ref.shape[0]
    dh = xa_ref.shape[1]
    wt = wt_ref[...]

    xa = xa_ref[...].astype(jnp.bfloat16)
    xb = xb_ref[...].astype(jnp.bfloat16)

    sa = jax.lax.dot_general(
        wt[:, :dh], xa, (((1,), (1,)), ((), ())),
        preferred_element_type=jnp.float32)
    sb = jax.lax.dot_general(
        wt[:, dh:], xb, (((1,), (1,)), ((), ())),
        preferred_element_type=jnp.float32)
    scores = sa + sb

    bm = jnp.max(scores).reshape(1, 1)
    m_old = m_ref[...]
    m_new = jnp.maximum(m_old, bm)
    alpha = jnp.exp(m_old - m_new)
    beta = jnp.exp(bm - m_new)

    e = jnp.exp(scores - bm)
    sub = jax.lax.broadcasted_iota(jnp.int32, (nseg, rows), 0)
    we = jnp.where(sub == ids, e, 0.0).astype(jnp.bfloat16)

    ones = jnp.ones((rows, 1), jnp.bfloat16)
    dsum = jax.lax.dot_general(
        we, ones, (((1,), (0,)), ((), ())),
        preferred_element_type=jnp.float32)
    ca = jax.lax.dot_general(
        we, xa, (((1,), (0,)), ((), ())),
        preferred_element_type=jnp.float32)
    cb = jax.lax.dot_general(
        we, xb, (((1,), (0,)), ((), ())),
        preferred_element_type=jnp.float32)

    m_ref[...] = m_new
    d_ref[...] = alpha * d_ref[...] + beta * dsum
    out_ref[:, :dh] = alpha * out_ref[:, :dh] + beta * ca
    out_ref[:, dh:] = alpha * out_ref[:, dh:] + beta * cb

    @pl.when(i == nb - 1)
    def _finish():
        d = d_ref[...]
        denom = jnp.where(d > 0.0, d, 1.0)
        out_ref[...] = out_ref[...] / denom


@jax.jit
def _attn_pool(x, segment_ids, W):
    n, d = x.shape
    rows = 2048
    nb = n // rows
    dh = d // 2
    ids = segment_ids.astype(jnp.int32).reshape(nb, 1, rows)
    wt = W.reshape(1, d).astype(jnp.bfloat16)
    return pl.pallas_call(
        functools.partial(_pool_kernel, nb=nb),
        grid=(nb,),
        in_specs=[
            pl.BlockSpec((1, 1, rows), lambda i: (i, 0, 0)),
            pl.BlockSpec((rows, dh), lambda i: (i, 0)),
            pl.BlockSpec((rows, dh), lambda i: (i, 1)),
            pl.BlockSpec((1, d), lambda i: (0, 0)),
        ],
        out_specs=pl.BlockSpec((_NSEG, d), lambda i: (0, 0)),
        out_shape=jax.ShapeDtypeStruct((_NSEG, d), jnp.float32),
        scratch_shapes=[
            pltpu.VMEM((1, 1), jnp.float32),
            pltpu.VMEM((_NSEG, 1), jnp.float32),
        ],
        compiler_params=pltpu.CompilerParams(
            dimension_semantics=("arbitrary",)),
    )(ids, x, x, wt)


def kernel(x, segment_ids, W):
    return _attn_pool(x, segment_ids, W)
